# baseline (speedup 1.0000x reference)
import functools

import jax
import jax.numpy as jnp
from jax import lax
from jax.experimental import pallas as pl
from jax.experimental.pallas import tpu as pltpu

_ROWS = 512
_TCHUNK = 4


def _ceil_to(n, m):
    return ((n + m - 1) // m) * m


def _lstm_tile(places_ref, x_ref, wih_ref, whh_ref, b_ref, out_ref, gin_ref,
               *, hidden):
    S, Bt, D = x_ref.shape
    H = hidden
    C = gin_ref.shape[0] // (2 * Bt)
    n_chunks = S // C

    whh = whh_ref[...]
    wih = wih_ref[...]
    bias = b_ref[...]
    places = places_ref[...]

    def project(chunk, slot):
        xc = x_ref[pl.ds(chunk * C, C)].reshape(C * Bt, D)
        gin_ref[pl.ds(slot * (C * Bt), C * Bt), :] = (
            jnp.dot(xc, wih, preferred_element_type=jnp.float32) + bias)

    def step(t, row, st):
        h, c, out = st
        gates = (jnp.dot(h, whh, preferred_element_type=jnp.float32)
                 + gin_ref[pl.ds(row, Bt), :])
        act = jax.nn.sigmoid(gates[:, 0:3 * H])
        i_g = act[:, 0 * H:1 * H]
        f_g = act[:, 1 * H:2 * H]
        o_g = act[:, 2 * H:3 * H]
        g_g = jnp.tanh(gates[:, 3 * H:4 * H])
        c = f_g * c + i_g * g_g
        h = o_g * jnp.tanh(c)
        out = jnp.where(places == t, h, out)
        return h, c, out

    def body(k, st):
        slot = lax.rem(k, 2)
        project(jnp.minimum(k + 1, n_chunks - 1), 1 - slot)
        base = slot * (C * Bt)
        for j in range(C):
            r = base + j * Bt
            st = step(k * C + j, r, st)
        return st

    project(0, 0)
    zeros = jnp.zeros((Bt, H), jnp.float32)
    _, _, out = lax.fori_loop(0, n_chunks, body, (zeros, zeros, zeros))
    out_ref[...] = out


@jax.jit
def kernel(x, real_positions, wih_packed, whh_packed, bias_packed):
    B, S, D = x.shape
    H, Gp = whh_packed.shape
    Bt = _ROWS
    Bp = _ceil_to(B, Bt)
    C = _TCHUNK if S % _TCHUNK == 0 else 1

    x_tm = jnp.transpose(x.astype(jnp.bfloat16), (1, 0, 2))
    if Bp != B:
        x_tm = jnp.pad(x_tm, ((0, 0), (0, Bp - B), (0, 0)))

    lengths = jnp.sum(real_positions.astype(jnp.float32), axis=-1)
    places = lengths.astype(jnp.int32) - 1
    places = jnp.where(places < 0, places + S, places)[:, None]
    if Bp != B:
        places = jnp.pad(places, ((0, Bp - B), (0, 0)))

    out = pl.pallas_call(
        functools.partial(_lstm_tile, hidden=H),
        out_shape=jax.ShapeDtypeStruct((Bp, H), jnp.float32),
        grid_spec=pltpu.PrefetchScalarGridSpec(
            num_scalar_prefetch=0,
            grid=(Bp // Bt,),
            in_specs=[
                pl.BlockSpec((Bt, 1), lambda g: (g, 0)),
                pl.BlockSpec((S, Bt, D), lambda g: (0, g, 0)),
                pl.BlockSpec((D, Gp), lambda g: (0, 0)),
                pl.BlockSpec((H, Gp), lambda g: (0, 0)),
                pl.BlockSpec((1, Gp), lambda g: (0, 0)),
            ],
            out_specs=pl.BlockSpec((Bt, H), lambda g: (g, 0)),
            scratch_shapes=[pltpu.VMEM((2 * C * Bt, Gp), jnp.float32)],
        ),
        compiler_params=pltpu.CompilerParams(
            dimension_semantics=("parallel",)),
    )(places, x_tm, wih_packed, whh_packed, bias_packed)

    return out[:B]


# speedup vs baseline: 1.3261x; 1.3261x over previous
import functools

import jax
import jax.numpy as jnp
from jax.experimental import pallas as pl
from jax.experimental.pallas import tpu as pltpu

_ROWS = 256
_TCHUNK = 16


def _ceil_to(n, m):
    return ((n + m - 1) // m) * m


def _lstm_tile(places_ref, x_ref, wih_ref, whh_ref, b_ref, out_ref,
               gin_ref, h_ref, c_ref, o_ref, *, hidden, n_chunks):
    Bt, C, D = x_ref.shape
    H = hidden
    k = pl.program_id(1)

    whh = whh_ref[...]
    bias = b_ref[...]
    places = places_ref[...]

    @pl.when(k == 0)
    def _init():
        h_ref[...] = jnp.zeros_like(h_ref)
        c_ref[...] = jnp.zeros_like(c_ref)
        o_ref[...] = jnp.zeros_like(o_ref)

    xt = jnp.swapaxes(x_ref[...], 0, 1).reshape(C * Bt, D)
    gin_ref[...] = (
        jnp.dot(xt.astype(jnp.bfloat16), wih_ref[...],
                preferred_element_type=jnp.float32) + bias)

    def step(t, row, st):
        h, c, out = st
        gates = (jnp.dot(h, whh, preferred_element_type=jnp.float32)
                 + gin_ref[pl.ds(row, Bt), :])
        act = jax.nn.sigmoid(gates[:, 0:3 * H])
        i_g = act[:, 0 * H:1 * H]
        f_g = act[:, 1 * H:2 * H]
        o_g = act[:, 2 * H:3 * H]
        g_g = jnp.tanh(gates[:, 3 * H:4 * H])
        c = f_g * c + i_g * g_g
        h = o_g * jnp.tanh(c)
        out = jnp.where(places == t, h, out)
        return h, c, out

    st = (h_ref[...], c_ref[...], o_ref[...])
    for j in range(C):
        st = step(k * C + j, pl.multiple_of(j * Bt, Bt), st)

    h_ref[...], c_ref[...], o_ref[...] = st
    out_ref[...] = st[2]


@jax.jit
def kernel(x, real_positions, wih_packed, whh_packed, bias_packed):
    B, S, D = x.shape
    H, Gp = whh_packed.shape
    Bt = _ROWS
    Bp = _ceil_to(B, Bt)
    C = _TCHUNK if S % _TCHUNK == 0 else S
    n_chunks = S // C

    x_p = x
    if Bp != B:
        x_p = jnp.pad(x, ((0, Bp - B), (0, 0), (0, 0)))

    lengths = jnp.sum(real_positions.astype(jnp.float32), axis=-1)
    places = lengths.astype(jnp.int32) - 1
    places = jnp.where(places < 0, places + S, places)[:, None]
    if Bp != B:
        places = jnp.pad(places, ((0, Bp - B), (0, 0)))

    out = pl.pallas_call(
        functools.partial(_lstm_tile, hidden=H, n_chunks=n_chunks),
        out_shape=jax.ShapeDtypeStruct((Bp, H), jnp.float32),
        grid_spec=pltpu.PrefetchScalarGridSpec(
            num_scalar_prefetch=0,
            grid=(Bp // Bt, n_chunks),
            in_specs=[
                pl.BlockSpec((Bt, 1), lambda g, k: (g, 0)),
                pl.BlockSpec((Bt, C, D), lambda g, k: (g, k, 0)),
                pl.BlockSpec((D, Gp), lambda g, k: (0, 0)),
                pl.BlockSpec((H, Gp), lambda g, k: (0, 0)),
                pl.BlockSpec((1, Gp), lambda g, k: (0, 0)),
            ],
            out_specs=pl.BlockSpec((Bt, H), lambda g, k: (g, 0)),
            scratch_shapes=[
                pltpu.VMEM((C * Bt, Gp), jnp.float32),
                pltpu.VMEM((Bt, H), jnp.float32),
                pltpu.VMEM((Bt, H), jnp.float32),
                pltpu.VMEM((Bt, H), jnp.float32),
            ],
        ),
        compiler_params=pltpu.CompilerParams(
            dimension_semantics=("parallel", "arbitrary")),
    )(places, x_p, wih_packed, whh_packed, bias_packed)

    return out[:B]
